# Initial kernel scaffold; baseline (speedup 1.0000x reference)
import jax
import jax.numpy as jnp
from jax.experimental import pallas as pl


def kernel(x, edge_index, W1, b1, W2, b2):
    raise NotImplementedError("write your pallas kernel here")


# speedup vs baseline: 28.4073x; 28.4073x over previous
import functools

import jax
import jax.numpy as jnp
from jax import lax
from jax.experimental import pallas as pl
from jax.experimental.pallas import tpu as pltpu
from jax.experimental.pallas import tpu_sc as plsc

N = 10000
E = 320000
D_IN = 128
D_HID = 16
N_CLASSES = 10

NC = 2
NS = 16
NW = NC * NS
L = 16

CHUNK = 128
K = 80
EP = NW * K * CHUNK
ROWS2D = EP // CHUNK

NP = 10112
RPT = NP // NS

_MESH = plsc.VectorSubcoreMesh(
    core_axis_name="c", subcore_axis_name="s", num_cores=NC, num_subcores=NS
)


def _zero_rows(buf, nrows):
    zero = jnp.zeros((L,), jnp.float32)

    def zb(i, c):
        buf[i, :] = zero
        return c

    lax.fori_loop(0, nrows, zb, 0)


@functools.partial(
    pl.kernel,
    out_type=jax.ShapeDtypeStruct((NC, NP, L), jnp.float32),
    mesh=_MESH,
    compiler_params=pltpu.CompilerParams(use_tc_tiling_on_sc=False),
    scratch_types=[
        pltpu.VMEM((RPT, L), jnp.float32),
        pltpu.VMEM((K, CHUNK), jnp.int32),
        pltpu.VMEM((CHUNK, L), jnp.float32),
        pltpu.VMEM_SHARED((NP, L), jnp.float32),
    ],
)
def _count_pass(dst2d, part, zbuf, dbuf, ones, acc):
    cid = lax.axis_index("c")
    sid = lax.axis_index("s")
    wid = sid * NC + cid

    _zero_rows(zbuf, RPT)
    pltpu.sync_copy(zbuf, acc.at[pl.ds(sid * RPT, RPT)])

    one = jnp.ones((L,), jnp.float32)

    def ob(i, c):
        ones[i, :] = one
        return c

    lax.fori_loop(0, CHUNK, ob, 0)

    pltpu.sync_copy(dst2d.at[pl.ds(wid * K, K)], dbuf)
    plsc.subcore_barrier()

    def chunk(j, c):
        pltpu.sync_copy(ones, acc.at[dbuf.at[j]], add=True)
        return c

    lax.fori_loop(0, K, chunk, 0)
    plsc.subcore_barrier()

    pltpu.sync_copy(acc.at[pl.ds(sid * RPT, RPT)], zbuf)
    pltpu.sync_copy(zbuf, part.at[cid, pl.ds(sid * RPT, RPT)])


@functools.partial(
    pl.kernel,
    out_type=jax.ShapeDtypeStruct((NC, NP, L), jnp.float32),
    mesh=_MESH,
    compiler_params=pltpu.CompilerParams(use_tc_tiling_on_sc=False),
    scratch_types=[
        pltpu.VMEM((RPT, L), jnp.float32),
        pltpu.VMEM((K, CHUNK), jnp.int32),
        pltpu.VMEM((K, CHUNK), jnp.int32),
        pltpu.VMEM((CHUNK, L), jnp.float32),
        pltpu.SemaphoreType.DMA,
        pltpu.VMEM_SHARED((NP, L), jnp.float32),
    ],
)
def _message_pass(table, src2d, dst2d, part, zbuf, sbuf, dbuf, rows, sem, acc):
    cid = lax.axis_index("c")
    sid = lax.axis_index("s")
    wid = sid * NC + cid

    _zero_rows(zbuf, RPT)
    pltpu.sync_copy(zbuf, acc.at[pl.ds(sid * RPT, RPT)])

    pltpu.sync_copy(src2d.at[pl.ds(wid * K, K)], sbuf)
    pltpu.sync_copy(dst2d.at[pl.ds(wid * K, K)], dbuf)
    plsc.subcore_barrier()

    def chunk(j, c):
        pltpu.async_copy(table.at[sbuf.at[j]], rows, sem).wait()
        pltpu.sync_copy(rows, acc.at[dbuf.at[j]], add=True)
        return c

    lax.fori_loop(0, K, chunk, 0)
    plsc.subcore_barrier()

    pltpu.sync_copy(acc.at[pl.ds(sid * RPT, RPT)], zbuf)
    pltpu.sync_copy(zbuf, part.at[cid, pl.ds(sid * RPT, RPT)])


def _t1_body(cnt_ref, x_ref, w1_ref, g1_ref, dis_ref):
    c = cnt_ref[0] + cnt_ref[1]
    dis = lax.rsqrt(c[:N, 0:1] + 1.0)
    h = jnp.dot(x_ref[...], w1_ref[...], preferred_element_type=jnp.float32)
    g1_ref[...] = h * dis
    dis_ref[...] = dis


def _t2_body(part_ref, g1_ref, dis_ref, b1_ref, r_ref):
    acc = part_ref[0][:N] + part_ref[1][:N] + g1_ref[...]
    out1 = acc * dis_ref[...] + b1_ref[...]
    r_ref[...] = jnp.maximum(out1, 0.0) * dis_ref[...]


def _t3_body(part_ref, r_ref, dis_ref, w2_ref, b2_ref, o_ref):
    acc = part_ref[0][:N] + part_ref[1][:N] + r_ref[...]
    t = acc * dis_ref[...]
    o_ref[...] = (
        jnp.dot(t, w2_ref[...], preferred_element_type=jnp.float32) + b2_ref[...]
    )


_t1 = pl.pallas_call(
    _t1_body,
    out_shape=[
        jax.ShapeDtypeStruct((N, D_HID), jnp.float32),
        jax.ShapeDtypeStruct((N, 1), jnp.float32),
    ],
)
_t2 = pl.pallas_call(
    _t2_body,
    out_shape=jax.ShapeDtypeStruct((N, D_HID), jnp.float32),
)
_t3 = pl.pallas_call(
    _t3_body,
    out_shape=jax.ShapeDtypeStruct((N, N_CLASSES), jnp.float32),
)


def kernel(x, edge_index, W1, b1, W2, b2):
    src = edge_index[0].astype(jnp.int32)
    dst = edge_index[1].astype(jnp.int32)
    pad = EP - E
    src2d = jnp.concatenate([src, jnp.zeros((pad,), jnp.int32)]).reshape(ROWS2D, CHUNK)
    dst2d = jnp.concatenate([dst, jnp.full((pad,), N, jnp.int32)]).reshape(ROWS2D, CHUNK)

    part_cnt = _count_pass(dst2d)
    g1, dis = _t1(part_cnt, x, W1)
    part1 = _message_pass(g1, src2d, dst2d)
    r = _t2(part1, g1, dis, b1.reshape(1, D_HID))
    part2 = _message_pass(r, src2d, dst2d)
    out = _t3(part2, r, dis, W2, b2.reshape(1, N_CLASSES))
    return out


# speedup vs baseline: 37.3400x; 1.3145x over previous
import functools

import jax
import jax.numpy as jnp
from jax import lax
from jax.experimental import pallas as pl
from jax.experimental.pallas import tpu as pltpu
from jax.experimental.pallas import tpu_sc as plsc

N = 10000
E = 320000
D_IN = 128
D_HID = 16
N_CLASSES = 10

NC = 2
NS = 16
NW = NC * NS
L = 16

CHUNK = 128
K = 80
EP = NW * K * CHUNK
ROWS2D = EP // CHUNK

NP = 10112
RPT = NP // NS

NBUF = 4
GROUPS = K // NBUF

_MESH = plsc.VectorSubcoreMesh(
    core_axis_name="c", subcore_axis_name="s", num_cores=NC, num_subcores=NS
)


def _zero_rows(buf, nrows):
    zero = jnp.zeros((L,), jnp.float32)

    def zb(i, c):
        buf[i, :] = zero
        return c

    lax.fori_loop(0, nrows, zb, 0)


@functools.partial(
    pl.kernel,
    out_type=jax.ShapeDtypeStruct((NC, NP, L), jnp.float32),
    mesh=_MESH,
    compiler_params=pltpu.CompilerParams(use_tc_tiling_on_sc=False),
    scratch_types=[
        pltpu.VMEM((RPT, L), jnp.float32),
        pltpu.VMEM((K, CHUNK), jnp.int32),
        pltpu.VMEM((CHUNK, L), jnp.float32),
        pltpu.SemaphoreType.DMA,
        pltpu.VMEM_SHARED((NP, L), jnp.float32),
    ],
)
def _count_pass(dst2d, part, zbuf, dbuf, ones, sem, acc):
    cid = lax.axis_index("c")
    sid = lax.axis_index("s")
    wid = sid * NC + cid

    _zero_rows(zbuf, RPT)
    pltpu.sync_copy(zbuf, acc.at[pl.ds(sid * RPT, RPT)])

    one = jnp.ones((L,), jnp.float32)

    def ob(i, c):
        ones[i, :] = one
        return c

    lax.fori_loop(0, CHUNK, ob, 0)

    pltpu.sync_copy(dst2d.at[pl.ds(wid * K, K)], dbuf)
    plsc.subcore_barrier()

    def fire(j, c):
        pltpu.async_copy(ones, acc.at[dbuf.at[j]], sem, add=True)
        return c

    lax.fori_loop(0, K, fire, 0)

    def drain(j, c):
        pltpu.make_async_copy(ones, acc.at[dbuf.at[0]], sem).wait()
        return c

    lax.fori_loop(0, K, drain, 0)
    plsc.subcore_barrier()

    pltpu.sync_copy(acc.at[pl.ds(sid * RPT, RPT)], zbuf)
    pltpu.sync_copy(zbuf, part.at[cid, pl.ds(sid * RPT, RPT)])


@functools.partial(
    pl.kernel,
    out_type=jax.ShapeDtypeStruct((NC, NP, L), jnp.float32),
    mesh=_MESH,
    compiler_params=pltpu.CompilerParams(use_tc_tiling_on_sc=False),
    scratch_types=[
        pltpu.VMEM((RPT, L), jnp.float32),
        pltpu.VMEM((K, CHUNK), jnp.int32),
        pltpu.VMEM((K, CHUNK), jnp.int32),
        pltpu.VMEM((NBUF * CHUNK, L), jnp.float32),
        pltpu.SemaphoreType.DMA,
        pltpu.SemaphoreType.DMA,
        pltpu.SemaphoreType.DMA,
        pltpu.SemaphoreType.DMA,
        pltpu.SemaphoreType.DMA,
        pltpu.VMEM_SHARED((NP, L), jnp.float32),
    ],
)
def _message_pass(table, src2d, dst2d, part, zbuf, sbuf, dbuf, rows,
                  sem0, sem1, sem2, sem3, isem, acc):
    cid = lax.axis_index("c")
    sid = lax.axis_index("s")
    wid = sid * NC + cid
    sems = [sem0, sem1, sem2, sem3]

    def rbuf(b):
        return rows.at[pl.ds(b * CHUNK, CHUNK)]

    pltpu.async_copy(src2d.at[pl.ds(wid * K, K)], sbuf, isem)
    pltpu.async_copy(dst2d.at[pl.ds(wid * K, K)], dbuf, isem)
    _zero_rows(zbuf, RPT)
    pltpu.sync_copy(zbuf, acc.at[pl.ds(sid * RPT, RPT)])
    pltpu.make_async_copy(src2d.at[pl.ds(wid * K, K)], sbuf, isem).wait()
    pltpu.make_async_copy(dst2d.at[pl.ds(wid * K, K)], dbuf, isem).wait()
    plsc.subcore_barrier()

    def fire_gather(j, b):
        pltpu.async_copy(table.at[sbuf.at[j]], rbuf(b), sems[b])

    def wait_gather(j, b):
        pltpu.make_async_copy(table.at[sbuf.at[j]], rbuf(b), sems[b]).wait()

    def fire_scatter(j, b):
        pltpu.async_copy(rbuf(b), acc.at[dbuf.at[j]], sems[b], add=True)

    def wait_scatter(j, b):
        pltpu.make_async_copy(rbuf(b), acc.at[dbuf.at[j]], sems[b]).wait()

    for b in range(NBUF):
        fire_gather(b, b)

    def group(i, c):
        base = i * NBUF
        for b in range(NBUF):
            wait_gather(base + b, b)
            fire_scatter(base + b, b)
        for b in range(NBUF):
            wait_scatter(base + b, b)
            fire_gather(base + b + NBUF, b)
        return c

    lax.fori_loop(0, GROUPS - 1, group, 0)

    base = (GROUPS - 1) * NBUF
    for b in range(NBUF):
        wait_gather(base + b, b)
        fire_scatter(base + b, b)
    for b in range(NBUF):
        wait_scatter(base + b, b)
    plsc.subcore_barrier()

    pltpu.sync_copy(acc.at[pl.ds(sid * RPT, RPT)], zbuf)
    pltpu.sync_copy(zbuf, part.at[cid, pl.ds(sid * RPT, RPT)])


def _t1_body(cnt_ref, x_ref, w1_ref, g1_ref, dis_ref):
    c = cnt_ref[0] + cnt_ref[1]
    dis = lax.rsqrt(c[:N, 0:1] + 1.0)
    h = jnp.dot(x_ref[...], w1_ref[...], preferred_element_type=jnp.float32)
    g1_ref[...] = h * dis
    dis_ref[...] = dis


def _t2_body(part_ref, g1_ref, dis_ref, b1_ref, r_ref):
    acc = part_ref[0][:N] + part_ref[1][:N] + g1_ref[...]
    out1 = acc * dis_ref[...] + b1_ref[...]
    r_ref[...] = jnp.maximum(out1, 0.0) * dis_ref[...]


def _t3_body(part_ref, r_ref, dis_ref, w2_ref, b2_ref, o_ref):
    acc = part_ref[0][:N] + part_ref[1][:N] + r_ref[...]
    t = acc * dis_ref[...]
    o_ref[...] = (
        jnp.dot(t, w2_ref[...], preferred_element_type=jnp.float32) + b2_ref[...]
    )


_t1 = pl.pallas_call(
    _t1_body,
    out_shape=[
        jax.ShapeDtypeStruct((N, D_HID), jnp.float32),
        jax.ShapeDtypeStruct((N, 1), jnp.float32),
    ],
)
_t2 = pl.pallas_call(
    _t2_body,
    out_shape=jax.ShapeDtypeStruct((N, D_HID), jnp.float32),
)
_t3 = pl.pallas_call(
    _t3_body,
    out_shape=jax.ShapeDtypeStruct((N, N_CLASSES), jnp.float32),
)


def kernel(x, edge_index, W1, b1, W2, b2):
    src = edge_index[0].astype(jnp.int32)
    dst = edge_index[1].astype(jnp.int32)
    pad = EP - E
    src2d = jnp.concatenate([src, jnp.zeros((pad,), jnp.int32)]).reshape(ROWS2D, CHUNK)
    dst2d = jnp.concatenate([dst, jnp.full((pad,), N, jnp.int32)]).reshape(ROWS2D, CHUNK)

    part_cnt = _count_pass(dst2d)
    g1, dis = _t1(part_cnt, x, W1)
    part1 = _message_pass(g1, src2d, dst2d)
    r = _t2(part1, g1, dis, b1.reshape(1, D_HID))
    part2 = _message_pass(r, src2d, dst2d)
    out = _t3(part2, r, dis, W2, b2.reshape(1, N_CLASSES))
    return out
